# baseline (speedup 1.0000x reference)
import functools

import jax
import jax.numpy as jnp
from jax import lax
from jax.experimental import pallas as pl
from jax.experimental.pallas import tpu as pltpu
from jax.experimental.pallas import tpu_sc as plsc

N_NODES = 10000
N_EDGES = 320000
N_GRAPHS = 16
EMB = 64

RN = 1000
GN = N_NODES // RN
RE = 6400
GE = N_EDGES // RE
RE2 = 8000
GE2 = N_EDGES // RE2

F32 = jnp.float32

NW = 32
EPW = N_EDGES // NW
SCC = 1000
NP1 = 10240
_SC_PARAMS = pltpu.CompilerParams(use_tc_tiling_on_sc=False)
_SC_MESH = dict(core_axis_name="c", subcore_axis_name="s")


_LOG2E = 1.4426950408889634
_LN2 = 0.6931471805599453


def _sp(x):
    y = jnp.maximum(-jnp.abs(x) * _LOG2E, -30.0)
    k = jnp.floor(y)
    f = y - k
    p = 0.0013333558146428441
    p = p * f + 0.0096181291075976
    p = p * f + 0.05550410866482158
    p = p * f + 0.2402265069591007
    p = p * f + 0.6931471805599453
    p = p * f + 1.0
    scale = lax.bitcast_convert_type((k.astype(jnp.int32) + 127) << 23, F32)
    z = p * scale
    s = z / (2.0 + z)
    s2 = s * s
    q = 1.0 / 9.0
    q = q * s2 + 1.0 / 7.0
    q = q * s2 + 1.0 / 5.0
    q = q * s2 + 1.0 / 3.0
    q = q * s2 + 1.0
    return 2.0 * s * q + jnp.maximum(x, 0.0)


def _pre_body(x_ref, wv_ref, bv_ref, ws_ref, wd_ref, st_ref, wu_ref, bu_ref,
              wue_ref, v0_ref, a_ref, b_ref, u0_ref, du_ref):
    i = pl.program_id(0)
    v0 = _sp(jnp.dot(x_ref[...], wv_ref[...], preferred_element_type=F32)
             + bv_ref[...])
    v0_ref[...] = v0
    a_ref[...] = jnp.dot(v0, ws_ref[...], preferred_element_type=F32)
    b_ref[...] = jnp.dot(v0, wd_ref[...], preferred_element_type=F32)

    @pl.when(i == 0)
    def _():
        u0 = _sp(jnp.dot(st_ref[...], wu_ref[...], preferred_element_type=F32)
                 + bu_ref[...])
        u0_ref[...] = u0
        du_ref[...] = jnp.dot(u0, wue_ref[...], preferred_element_type=F32)


def _pre_call(x, wv, bv, ws, wd, st, wu, bu, wue):
    d_node = x.shape[1]
    d_u = st.shape[1]
    full = lambda shape: pl.BlockSpec(shape, lambda i: (0,) * len(shape))
    return pl.pallas_call(
        _pre_body,
        grid=(GN,),
        in_specs=[
            pl.BlockSpec((RN, d_node), lambda i: (i, 0)),
            full((d_node, EMB)), full((1, EMB)), full((EMB, EMB)),
            full((EMB, EMB)), full((N_GRAPHS, d_u)), full((d_u, EMB)),
            full((1, EMB)), full((EMB, EMB)),
        ],
        out_specs=[
            pl.BlockSpec((RN, EMB), lambda i: (i, 0)),
            pl.BlockSpec((RN, EMB), lambda i: (i, 0)),
            pl.BlockSpec((RN, EMB), lambda i: (i, 0)),
            pl.BlockSpec((N_GRAPHS, EMB), lambda i: (0, 0)),
            pl.BlockSpec((N_GRAPHS, EMB), lambda i: (0, 0)),
        ],
        out_shape=[
            jax.ShapeDtypeStruct((N_NODES, EMB), F32),
            jax.ShapeDtypeStruct((N_NODES, EMB), F32),
            jax.ShapeDtypeStruct((N_NODES, EMB), F32),
            jax.ShapeDtypeStruct((N_GRAPHS, EMB), F32),
            jax.ShapeDtypeStruct((N_GRAPHS, EMB), F32),
        ],
    )(x, wv, bv, ws, wd, st, wu, bu, wue)


RF = RE // 2
GF = N_EDGES // 2 // RF


def _edge_half(ea_blk, bb_row, g, we1_ref, be1_ref, we_ref, beu_ref, du_ref,
               we2_ref, sc_ref):
    e0 = _sp(lax.dot_general(ea_blk, we1_ref[...], (((0,), (0,)), ((), ())),
                             preferred_element_type=F32) + be1_ref[...])
    oh = (bb_row[:, None]
          == lax.broadcasted_iota(jnp.int32, (RF, N_GRAPHS), 1)).astype(F32)
    dub = jnp.dot(oh, du_ref[...], preferred_element_type=F32)
    e1 = _sp(g + jnp.dot(e0, we_ref[...], preferred_element_type=F32)
             + dub + beu_ref[...])
    eB = e1 + e0
    e0p = _sp(jnp.sum(eB * we2_ref[...], axis=1) + sc_ref[0, 0])
    e1c = jnp.concatenate([e1, jnp.ones((RF, EMB), F32)], axis=1)
    seg = lax.dot_general(oh, e1c, (((0,), (0,)), ((), ())),
                          preferred_element_type=F32)
    return e1, e0p, seg


def _edge_body(ea_lo_ref, ea_hi_ref, ga_ref, gb_ref, bb_lo_ref, bb_hi_ref,
               we1_ref, be1_ref, we_ref, beu_ref, du_ref, we2_ref, sc_ref,
               e1_ref, e0p_lo_ref, e0p_hi_ref, acc_ref):
    i = pl.program_id(0)
    gab = ga_ref[...] + gb_ref[...]
    e1_lo, e0p_lo, seg_lo = _edge_half(
        ea_lo_ref[...], bb_lo_ref[0, 0, :], gab[:, :EMB], we1_ref, be1_ref,
        we_ref, beu_ref, du_ref, we2_ref, sc_ref)
    e1_hi, e0p_hi, seg_hi = _edge_half(
        ea_hi_ref[...], bb_hi_ref[0, 0, :], gab[:, EMB:], we1_ref, be1_ref,
        we_ref, beu_ref, du_ref, we2_ref, sc_ref)
    e1_ref[...] = jnp.concatenate([e1_lo, e1_hi], axis=1)
    e0p_lo_ref[0, 0, :] = e0p_lo
    e0p_hi_ref[0, 0, :] = e0p_hi

    @pl.when(i == 0)
    def _():
        acc_ref[...] = jnp.zeros_like(acc_ref)

    acc_ref[...] += seg_lo + seg_hi


def _edge_call(ea_t, ga, gb, bb3, we1, be1, we, beu, du, we2r, sc):
    d_edge = ea_t.shape[0]
    full = lambda shape: pl.BlockSpec(shape, lambda i: (0,) * len(shape))
    return pl.pallas_call(
        _edge_body,
        grid=(GF,),
        in_specs=[
            pl.BlockSpec((d_edge, RF), lambda i: (0, i)),
            pl.BlockSpec((d_edge, RF), lambda i: (0, i + GF)),
            pl.BlockSpec((RF, 2 * EMB), lambda i: (i, 0)),
            pl.BlockSpec((RF, 2 * EMB), lambda i: (i, 0)),
            pl.BlockSpec((1, 1, RF), lambda i: (i, 0, 0)),
            pl.BlockSpec((1, 1, RF), lambda i: (i + GF, 0, 0)),
            full((d_edge, EMB)), full((1, EMB)), full((EMB, EMB)),
            full((1, EMB)), full((N_GRAPHS, EMB)), full((1, EMB)),
            full((1, 8)),
        ],
        out_specs=[
            pl.BlockSpec((RF, 2 * EMB), lambda i: (i, 0)),
            pl.BlockSpec((1, 1, RF), lambda i: (i, 0, 0)),
            pl.BlockSpec((1, 1, RF), lambda i: (i, 0, 0)),
            pl.BlockSpec((N_GRAPHS, 2 * EMB), lambda i: (0, 0)),
        ],
        out_shape=[
            jax.ShapeDtypeStruct((N_EDGES // 2, 2 * EMB), F32),
            jax.ShapeDtypeStruct((GF, 1, RF), F32),
            jax.ShapeDtypeStruct((GF, 1, RF), F32),
            jax.ShapeDtypeStruct((N_GRAPHS, 2 * EMB), F32),
        ],
    )(ea_t, ea_t, ga, gb, bb3, bb3, we1, be1, we, beu, du, we2r, sc)


def _node_body(v0_ref, sp_ref, cnt_ref, bt_ref, u0_ref, wvv_ref, wve_ref,
               wvu_ref, bvu_ref, wv2_ref, eacc_ref, wuv_ref, wue_ref,
               wuu_ref, buu_ref, wu2_ref, sc_ref,
               v0p_ref, sa_ref, sb_ref, u0p_ref, vacc_ref):
    i = pl.program_id(0)
    cnt = cnt_ref[0, 0, :] + cnt_ref[0, 1, :]
    inv = 1.0 / jnp.maximum(cnt, 1.0)
    e_to_v = (sp_ref[0] + sp_ref[1]) * inv[:, None]
    btv = bt_ref[0, 0, :]
    oh = (btv[:, None] == lax.broadcasted_iota(jnp.int32, (RN, N_GRAPHS), 1)
          ).astype(F32)
    ub = jnp.dot(u0_ref[...], wvu_ref[...], preferred_element_type=F32)
    v0 = v0_ref[...]
    v1 = _sp(jnp.dot(v0, wvv_ref[...], preferred_element_type=F32)
             + jnp.dot(e_to_v, wve_ref[...], preferred_element_type=F32)
             + jnp.dot(oh, ub, preferred_element_type=F32) + bvu_ref[...])
    v0p = _sp(jnp.sum((v1 + v0) * wv2_ref[...], axis=1) + sc_ref[0, 0])
    v0p_ref[0, 0, :] = v0p
    sa_ref[0, 0, :] = sc_ref[0, 2] * v0p
    sb_ref[0, 0, :] = sc_ref[0, 3] * v0p

    @pl.when(i == 0)
    def _():
        vacc_ref[...] = jnp.zeros_like(vacc_ref)

    seg = lax.dot_general(oh, v1, (((0,), (0,)), ((), ())),
                          preferred_element_type=F32)
    vacc_ref[:, :EMB] += seg
    vacc_ref[:, EMB:] += jnp.sum(oh, axis=0)[:, None]

    @pl.when(i == pl.num_programs(0) - 1)
    def _():
        v_to_u = vacc_ref[:, :EMB] / jnp.maximum(vacc_ref[:, EMB:], 1.0)
        e_to_u = eacc_ref[:, :EMB] / jnp.maximum(eacc_ref[:, EMB:], 1.0)
        u0 = u0_ref[...]
        u1 = _sp(jnp.dot(v_to_u, wuv_ref[...], preferred_element_type=F32)
                 + jnp.dot(e_to_u, wue_ref[...], preferred_element_type=F32)
                 + jnp.dot(u0, wuu_ref[...], preferred_element_type=F32)
                 + buu_ref[...])
        u0p_ref[0, :] = _sp(jnp.sum((u1 + u0) * wu2_ref[...], axis=1)
                            + sc_ref[0, 1])


def _node_call(v0, sp_, cntp, bt3, u0, wvv, wve, wvu, bvu, wv2r, eacc,
               wuv, wue, wuu, buu, wu2r, sc):
    full = lambda shape: pl.BlockSpec(shape, lambda i: (0,) * len(shape))
    return pl.pallas_call(
        _node_body,
        grid=(GN,),
        in_specs=[
            pl.BlockSpec((RN, EMB), lambda i: (i, 0)),
            pl.BlockSpec((2, RN, EMB), lambda i: (0, i, 0)),
            pl.BlockSpec((1, 2, RN), lambda i: (i, 0, 0)),
            pl.BlockSpec((1, 1, RN), lambda i: (i, 0, 0)),
            full((N_GRAPHS, EMB)), full((EMB, EMB)), full((EMB, EMB)),
            full((EMB, EMB)), full((1, EMB)), full((1, EMB)),
            full((N_GRAPHS, 2 * EMB)), full((EMB, EMB)), full((EMB, EMB)),
            full((EMB, EMB)), full((1, EMB)), full((1, EMB)), full((1, 8)),
        ],
        out_specs=[
            pl.BlockSpec((1, 1, RN), lambda i: (i, 0, 0)),
            pl.BlockSpec((1, 1, RN), lambda i: (i, 0, 0)),
            pl.BlockSpec((1, 1, RN), lambda i: (i, 0, 0)),
            pl.BlockSpec((1, N_GRAPHS), lambda i: (0, 0)),
        ],
        out_shape=[
            jax.ShapeDtypeStruct((GN, 1, RN), F32),
            jax.ShapeDtypeStruct((GN, 1, RN), F32),
            jax.ShapeDtypeStruct((GN, 1, RN), F32),
            jax.ShapeDtypeStruct((1, N_GRAPHS), F32),
        ],
        scratch_shapes=[pltpu.VMEM((N_GRAPHS, 2 * EMB), F32)],
    )(v0, sp_, cntp, bt3, u0, wvv, wve, wvu, bvu, wv2r, eacc,
      wuv, wue, wuu, buu, wu2r, sc)


E2R = 2500
E2B = 2500


def _edge2_body(ts_ref, td_ref, e0p_ref, bb_ref, u0p_ref, sc_ref, e1p_ref,
                eo_ref):
    bbv = bb_ref[...]
    u_e = jnp.zeros((E2B, 128), F32)
    for g in range(N_GRAPHS):
        u_e = u_e + jnp.where(bbv == g, u0p_ref[0, g], 0.0)
    e0p = e0p_ref[...]
    we, wu, beu2 = sc_ref[0, 0], sc_ref[0, 1], sc_ref[0, 2]
    e1p = _sp(ts_ref[...] + td_ref[...] + we * e0p + wu * u_e + beu2)
    e1p_ref[...] = e1p
    eo_ref[...] = e1p + e0p


def _edge2_call(ts2, td2, e0p2, bb2, u0p, sc):
    full = lambda shape: pl.BlockSpec(shape, lambda i: (0,) * len(shape))
    blk = lambda: pl.BlockSpec((E2B, 128), lambda i: (i, 0))
    return pl.pallas_call(
        _edge2_body,
        grid=(E2R // E2B,),
        in_specs=[blk(), blk(), blk(), blk(), full((1, N_GRAPHS)),
                  full((1, 8))],
        out_specs=[blk(), blk()],
        out_shape=[
            jax.ShapeDtypeStruct((E2R, 128), F32),
            jax.ShapeDtypeStruct((E2R, 128), F32),
        ],
    )(ts2, td2, e0p2, bb2, u0p, sc)


def _node2_body(v0p_ref, s2_ref, cnt_ref, bt_ref, u0p_ref, sc_ref, vo_ref):
    cnt = cnt_ref[0, 0, :] + cnt_ref[0, 1, :]
    inv = 1.0 / jnp.maximum(cnt, 1.0)
    e_to_vp = (s2_ref[0, 0, :] + s2_ref[0, 1, :]) * inv
    btv = bt_ref[0, 0, :]
    oh = (btv[:, None] == lax.broadcasted_iota(jnp.int32, (RN, N_GRAPHS), 1)
          ).astype(F32)
    u_n = jnp.sum(oh * u0p_ref[...], axis=1)
    wvv, wve, wvu, bv2u = (sc_ref[0, 0], sc_ref[0, 1], sc_ref[0, 2],
                           sc_ref[0, 3])
    v0p = v0p_ref[0, 0, :]
    v1p = _sp(wvv * v0p + wve * e_to_vp + wvu * u_n + bv2u)
    vo_ref[0, 0, :] = v1p + v0p


def _node2_call(v0p3, s2p, cntp, bt3, u0p, sc):
    full = lambda shape: pl.BlockSpec(shape, lambda i: (0,) * len(shape))
    return pl.pallas_call(
        _node2_body,
        grid=(GN,),
        in_specs=[
            pl.BlockSpec((1, 1, RN), lambda i: (i, 0, 0)),
            pl.BlockSpec((1, 2, RN), lambda i: (i, 0, 0)),
            pl.BlockSpec((1, 2, RN), lambda i: (i, 0, 0)),
            pl.BlockSpec((1, 1, RN), lambda i: (i, 0, 0)),
            full((1, N_GRAPHS)), full((1, 8)),
        ],
        out_specs=pl.BlockSpec((1, 1, RN), lambda i: (i, 0, 0)),
        out_shape=jax.ShapeDtypeStruct((GN, 1, RN), F32),
    )(v0p3, s2p, cntp, bt3, u0p, sc)


def _sc_gather_body(a_hbm, b_hbm, src_hbm, dst_hbm, ga_hbm, gb_hbm,
                    idx_v, buf_v, sem):
    wid = lax.axis_index("s") * 2 + lax.axis_index("c")
    half = wid // 16
    rbase = (wid % 16) * EPW
    col = half * EMB
    base = wid * EPW
    for kk in range(EPW // SCC):
        off = base + kk * SCC
        ro = rbase + kk * SCC
        pltpu.sync_copy(src_hbm.at[pl.ds(off, SCC)], idx_v)
        pltpu.async_copy(a_hbm.at[idx_v], buf_v, sem).wait()
        pltpu.sync_copy(buf_v, ga_hbm.at[pl.ds(ro, SCC), pl.ds(col, EMB)])
        pltpu.sync_copy(dst_hbm.at[pl.ds(off, SCC)], idx_v)
        pltpu.async_copy(b_hbm.at[idx_v], buf_v, sem).wait()
        pltpu.sync_copy(buf_v, gb_hbm.at[pl.ds(ro, SCC), pl.ds(col, EMB)])


_sc_gather = pl.kernel(
    _sc_gather_body,
    mesh=plsc.VectorSubcoreMesh(**_SC_MESH),
    compiler_params=_SC_PARAMS,
    out_type=[jax.ShapeDtypeStruct((N_EDGES // 2, 2 * EMB), F32),
              jax.ShapeDtypeStruct((N_EDGES // 2, 2 * EMB), F32)],
    scratch_types=[pltpu.VMEM((SCC,), jnp.int32),
                   pltpu.VMEM((SCC, EMB), F32),
                   pltpu.SemaphoreType.DMA],
)


SCF = 1000


def _sc_scatter_body(e1_hbm, dst_hbm, z2_hbm, z1_hbm, one_hbm, s_hbm,
                     cnt_hbm, idx_v, buf_v, ones_v, acc, acc1, sem):
    c = lax.axis_index("c")
    s = lax.axis_index("s")
    wid = s * 2 + c
    rpt = N_NODES // 16
    rpt1 = NP1 // 16
    half = N_EDGES // 2
    pltpu.sync_copy(z2_hbm.at[pl.ds(s * rpt, rpt)],
                    acc.at[pl.ds(s * rpt, rpt)])
    pltpu.sync_copy(z1_hbm.at[pl.ds(s * rpt1, rpt1)],
                    acc1.at[pl.ds(s * rpt1, rpt1)])
    pltpu.sync_copy(one_hbm, ones_v)
    plsc.subcore_barrier()
    fbase = wid * (EPW // 2)
    for kk in range(EPW // 2 // SCF):
        ro = fbase + kk * SCF
        for col, ebase in ((0, 0), (EMB, half)):
            pltpu.sync_copy(dst_hbm.at[pl.ds(ebase + ro, SCF)], idx_v)
            pltpu.sync_copy(e1_hbm.at[pl.ds(ro, SCF), pl.ds(col, EMB)],
                            buf_v)
            pltpu.sync_copy(buf_v, acc.at[idx_v], add=True)
            pltpu.sync_copy(ones_v, acc1.at[idx_v], add=True)
    plsc.subcore_barrier()
    pltpu.sync_copy(acc.at[pl.ds(s * rpt, rpt)],
                    s_hbm.at[c, pl.ds(s * rpt, rpt)])
    pltpu.sync_copy(acc1.at[pl.ds(s * rpt1, rpt1)],
                    cnt_hbm.at[c, pl.ds(s * rpt1, rpt1)])


_sc_scatter = pl.kernel(
    _sc_scatter_body,
    mesh=plsc.VectorSubcoreMesh(**_SC_MESH),
    compiler_params=_SC_PARAMS,
    out_type=[jax.ShapeDtypeStruct((2, N_NODES, EMB), F32),
              jax.ShapeDtypeStruct((2, NP1), F32)],
    scratch_types=[pltpu.VMEM((SCF,), jnp.int32),
                   pltpu.VMEM((SCF, EMB), F32),
                   pltpu.VMEM((SCF,), F32),
                   pltpu.VMEM_SHARED((N_NODES, EMB), F32),
                   pltpu.VMEM_SHARED((NP1,), F32),
                   pltpu.SemaphoreType.DMA],
)


def _sc_gather1_body(sa_hbm, sb_hbm, src_hbm, dst_hbm, ts_hbm, td_hbm,
                     idx_v, val_v, sem):
    wid = lax.axis_index("s") * 2 + lax.axis_index("c")
    base = wid * EPW
    for kk in range(EPW // SCC):
        off = base + kk * SCC
        pltpu.sync_copy(src_hbm.at[pl.ds(off, SCC)], idx_v)
        pltpu.async_copy(sa_hbm.at[idx_v], val_v, sem).wait()
        pltpu.sync_copy(val_v, ts_hbm.at[pl.ds(off, SCC)])
        pltpu.sync_copy(dst_hbm.at[pl.ds(off, SCC)], idx_v)
        pltpu.async_copy(sb_hbm.at[idx_v], val_v, sem).wait()
        pltpu.sync_copy(val_v, td_hbm.at[pl.ds(off, SCC)])


_sc_gather1 = pl.kernel(
    _sc_gather1_body,
    mesh=plsc.VectorSubcoreMesh(**_SC_MESH),
    compiler_params=_SC_PARAMS,
    out_type=[jax.ShapeDtypeStruct((N_EDGES,), F32),
              jax.ShapeDtypeStruct((N_EDGES,), F32)],
    scratch_types=[pltpu.VMEM((SCC,), jnp.int32),
                   pltpu.VMEM((SCC,), F32),
                   pltpu.SemaphoreType.DMA],
)


def _sc_scatter1_body(ep_hbm, dst_hbm, z1_hbm, s2_hbm, idx_v, val_v, acc1,
                      sem):
    c = lax.axis_index("c")
    s = lax.axis_index("s")
    wid = s * 2 + c
    rpt1 = NP1 // 16
    pltpu.sync_copy(z1_hbm.at[pl.ds(s * rpt1, rpt1)],
                    acc1.at[pl.ds(s * rpt1, rpt1)])
    plsc.subcore_barrier()
    base = wid * EPW
    for kk in range(EPW // SCC):
        off = base + kk * SCC
        pltpu.sync_copy(dst_hbm.at[pl.ds(off, SCC)], idx_v)
        pltpu.sync_copy(ep_hbm.at[pl.ds(off, SCC)], val_v)
        pltpu.sync_copy(val_v, acc1.at[idx_v], add=True)
    plsc.subcore_barrier()
    pltpu.sync_copy(acc1.at[pl.ds(s * rpt1, rpt1)],
                    s2_hbm.at[c, pl.ds(s * rpt1, rpt1)])


_sc_scatter1 = pl.kernel(
    _sc_scatter1_body,
    mesh=plsc.VectorSubcoreMesh(**_SC_MESH),
    compiler_params=_SC_PARAMS,
    out_type=jax.ShapeDtypeStruct((2, NP1), F32),
    scratch_types=[pltpu.VMEM((SCC,), jnp.int32),
                   pltpu.VMEM((SCC,), F32),
                   pltpu.VMEM_SHARED((NP1,), F32),
                   pltpu.SemaphoreType.DMA],
)


def kernel(x, edge_index, edge_attr, state, batch, bond_batch, params1,
           params2):
    p1, p2 = params1, params2
    src = edge_index[0].astype(jnp.int32)
    dst = edge_index[1].astype(jnp.int32)
    batch_i = batch.astype(jnp.int32)
    bb_i = bond_batch.astype(jnp.int32)

    row = lambda v: v.reshape(1, -1)

    W = p1["We_upd"]
    v0, a, b, u0, du = _pre_call(
        x, p1["Wv_pre"], row(p1["bv_pre"]), W[:EMB], W[EMB:2 * EMB],
        state, p1["Wu_pre"], row(p1["bu_pre"]), W[3 * EMB:])

    ga, gb = _sc_gather(a, b, src, dst)

    sc_c = jnp.zeros((1, 8), F32).at[0, 0].set(p2["be_pre"][0])
    bb3 = bb_i.reshape(2 * GF, 1, RF)
    e1f, e0p_lo3, e0p_hi3, eacc = _edge_call(
        edge_attr.T, ga, gb, bb3, p1["We_pre"], row(p1["be_pre"]),
        W[2 * EMB:3 * EMB], row(p1["be_upd"]), du, row(p2["We_pre"][:, 0]),
        sc_c)

    half = N_EDGES // 2
    z2 = jnp.zeros((N_NODES, EMB), F32)
    z1 = jnp.zeros((NP1,), F32)
    one = jnp.ones((SCF,), F32)
    sp_, cntp = _sc_scatter(e1f, dst, z2, z1, one)
    cntp3 = cntp[:, :N_NODES].reshape(2, GN, RN).transpose(1, 0, 2)

    Wv = p1["Wv_upd"]
    Wu = p1["Wu_upd"]
    W2 = p2["We_upd"][:, 0]
    sc_e = (jnp.zeros((1, 8), F32).at[0, 0].set(p2["bv_pre"][0])
            .at[0, 1].set(p2["bu_pre"][0])
            .at[0, 2].set(W2[0]).at[0, 3].set(W2[1]))
    bt3 = batch_i.reshape(GN, 1, RN)
    v0p3, sa3, sb3, u0p = _node_call(
        v0, sp_, cntp3, bt3, u0, Wv[:EMB], Wv[EMB:2 * EMB], Wv[2 * EMB:],
        row(p1["bv_upd"]), row(p2["Wv_pre"][:, 0]), eacc,
        Wu[:EMB], Wu[EMB:2 * EMB], Wu[2 * EMB:], row(p1["bu_upd"]),
        row(p2["Wu_pre"][:, 0]), sc_e)

    ts, td = _sc_gather1(sa3.reshape(N_NODES), sb3.reshape(N_NODES),
                         src, dst)

    sc_g = (jnp.zeros((1, 8), F32).at[0, 0].set(W2[2]).at[0, 1].set(W2[3])
            .at[0, 2].set(p2["be_upd"][0]))
    e0p_f = jnp.concatenate([e0p_lo3.reshape(half), e0p_hi3.reshape(half)])
    e1p2, eo2 = _edge2_call(ts.reshape(E2R, 128),
                            td.reshape(E2R, 128),
                            e0p_f.reshape(E2R, 128),
                            bb_i.reshape(E2R, 128), u0p, sc_g)

    s2p = _sc_scatter1(e1p2.reshape(N_EDGES), dst, z1)
    s2p3 = s2p[:, :N_NODES].reshape(2, GN, RN).transpose(1, 0, 2)

    Wv2 = p2["Wv_upd"][:, 0]
    sc_i = (jnp.zeros((1, 8), F32).at[0, 0].set(Wv2[0]).at[0, 1].set(Wv2[1])
            .at[0, 2].set(Wv2[2]).at[0, 3].set(p2["bv_upd"][0]))
    vo3 = _node2_call(v0p3, s2p3, cntp3, bt3, u0p, sc_i)

    v_out = vo3.reshape(N_NODES, 1)
    e_out = eo2.reshape(N_EDGES, 1)
    return (v_out, e_out, edge_index)


# speedup vs baseline: 1.4772x; 1.4772x over previous
import functools

import jax
import jax.numpy as jnp
from jax import lax
from jax.experimental import pallas as pl
from jax.experimental.pallas import tpu as pltpu
from jax.experimental.pallas import tpu_sc as plsc

N_NODES = 10000
N_EDGES = 320000
N_GRAPHS = 16
EMB = 64

RN = 1000
GN = N_NODES // RN
RE = 6400
GE = N_EDGES // RE
RE2 = 8000
GE2 = N_EDGES // RE2

F32 = jnp.float32

NW = 32
EPW = N_EDGES // NW
SCC = 1000
NP1 = 10240
_SC_PARAMS = pltpu.CompilerParams(use_tc_tiling_on_sc=False)
_SC_MESH = dict(core_axis_name="c", subcore_axis_name="s")


_LOG2E = 1.4426950408889634
_LN2 = 0.6931471805599453


def _sp(x):
    return _LN2 * jnp.log2(1.0 + jnp.exp2(-_LOG2E * jnp.abs(x))) \
        + jnp.maximum(x, 0.0)


def _pre_body(x_ref, wv_ref, bv_ref, ws_ref, wd_ref, st_ref, wu_ref, bu_ref,
              wue_ref, v0_ref, a_ref, b_ref, u0_ref, du_ref):
    i = pl.program_id(0)
    v0 = _sp(jnp.dot(x_ref[...], wv_ref[...], preferred_element_type=F32)
             + bv_ref[...])
    v0_ref[...] = v0
    a_ref[...] = jnp.dot(v0, ws_ref[...], preferred_element_type=F32)
    b_ref[...] = jnp.dot(v0, wd_ref[...], preferred_element_type=F32)

    @pl.when(i == 0)
    def _():
        u0 = _sp(jnp.dot(st_ref[...], wu_ref[...], preferred_element_type=F32)
                 + bu_ref[...])
        u0_ref[...] = u0
        du_ref[...] = jnp.dot(u0, wue_ref[...], preferred_element_type=F32)


def _pre_call(x, wv, bv, ws, wd, st, wu, bu, wue):
    d_node = x.shape[1]
    d_u = st.shape[1]
    full = lambda shape: pl.BlockSpec(shape, lambda i: (0,) * len(shape))
    return pl.pallas_call(
        _pre_body,
        grid=(GN,),
        in_specs=[
            pl.BlockSpec((RN, d_node), lambda i: (i, 0)),
            full((d_node, EMB)), full((1, EMB)), full((EMB, EMB)),
            full((EMB, EMB)), full((N_GRAPHS, d_u)), full((d_u, EMB)),
            full((1, EMB)), full((EMB, EMB)),
        ],
        out_specs=[
            pl.BlockSpec((RN, EMB), lambda i: (i, 0)),
            pl.BlockSpec((RN, EMB), lambda i: (i, 0)),
            pl.BlockSpec((RN, EMB), lambda i: (i, 0)),
            pl.BlockSpec((N_GRAPHS, EMB), lambda i: (0, 0)),
            pl.BlockSpec((N_GRAPHS, EMB), lambda i: (0, 0)),
        ],
        out_shape=[
            jax.ShapeDtypeStruct((N_NODES, EMB), F32),
            jax.ShapeDtypeStruct((N_NODES, EMB), F32),
            jax.ShapeDtypeStruct((N_NODES, EMB), F32),
            jax.ShapeDtypeStruct((N_GRAPHS, EMB), F32),
            jax.ShapeDtypeStruct((N_GRAPHS, EMB), F32),
        ],
    )(x, wv, bv, ws, wd, st, wu, bu, wue)


RF = RE // 2
GF = N_EDGES // 2 // RF


def _edge_half(ea_blk, bb_row, g, we1_ref, be1_ref, we_ref, beu_ref, du_ref,
               we2_ref, sc_ref):
    e0 = _sp(lax.dot_general(ea_blk, we1_ref[...], (((0,), (0,)), ((), ())),
                             preferred_element_type=F32) + be1_ref[...])
    oh = (bb_row[:, None]
          == lax.broadcasted_iota(jnp.int32, (RF, N_GRAPHS), 1)).astype(F32)
    dub = jnp.dot(oh, du_ref[...], preferred_element_type=F32)
    e1 = _sp(g + jnp.dot(e0, we_ref[...], preferred_element_type=F32)
             + dub + beu_ref[...])
    eB = e1 + e0
    e0p = _sp(jnp.sum(eB * we2_ref[...], axis=1) + sc_ref[0, 0])
    e1c = jnp.concatenate([e1, jnp.ones((RF, EMB), F32)], axis=1)
    seg = lax.dot_general(oh, e1c, (((0,), (0,)), ((), ())),
                          preferred_element_type=F32)
    return e1, e0p, seg


def _edge_body(ea_lo_ref, ea_hi_ref, ga_ref, gb_ref, bb_lo_ref, bb_hi_ref,
               we1_ref, be1_ref, we_ref, beu_ref, du_ref, we2_ref, sc_ref,
               e1_ref, e0p_lo_ref, e0p_hi_ref, acc_ref):
    i = pl.program_id(0)
    gab = ga_ref[...] + gb_ref[...]
    e1_lo, e0p_lo, seg_lo = _edge_half(
        ea_lo_ref[...], bb_lo_ref[0, 0, :], gab[:, :EMB], we1_ref, be1_ref,
        we_ref, beu_ref, du_ref, we2_ref, sc_ref)
    e1_hi, e0p_hi, seg_hi = _edge_half(
        ea_hi_ref[...], bb_hi_ref[0, 0, :], gab[:, EMB:], we1_ref, be1_ref,
        we_ref, beu_ref, du_ref, we2_ref, sc_ref)
    e1_ref[...] = jnp.concatenate([e1_lo, e1_hi], axis=1)
    e0p_lo_ref[0, 0, :] = e0p_lo
    e0p_hi_ref[0, 0, :] = e0p_hi

    @pl.when(i == 0)
    def _():
        acc_ref[...] = jnp.zeros_like(acc_ref)

    acc_ref[...] += seg_lo + seg_hi


def _edge_call(ea_t, ga, gb, bb3, we1, be1, we, beu, du, we2r, sc):
    d_edge = ea_t.shape[0]
    full = lambda shape: pl.BlockSpec(shape, lambda i: (0,) * len(shape))
    return pl.pallas_call(
        _edge_body,
        grid=(GF,),
        in_specs=[
            pl.BlockSpec((d_edge, RF), lambda i: (0, i)),
            pl.BlockSpec((d_edge, RF), lambda i: (0, i + GF)),
            pl.BlockSpec((RF, 2 * EMB), lambda i: (i, 0)),
            pl.BlockSpec((RF, 2 * EMB), lambda i: (i, 0)),
            pl.BlockSpec((1, 1, RF), lambda i: (i, 0, 0)),
            pl.BlockSpec((1, 1, RF), lambda i: (i + GF, 0, 0)),
            full((d_edge, EMB)), full((1, EMB)), full((EMB, EMB)),
            full((1, EMB)), full((N_GRAPHS, EMB)), full((1, EMB)),
            full((1, 8)),
        ],
        out_specs=[
            pl.BlockSpec((RF, 2 * EMB), lambda i: (i, 0)),
            pl.BlockSpec((1, 1, RF), lambda i: (i, 0, 0)),
            pl.BlockSpec((1, 1, RF), lambda i: (i, 0, 0)),
            pl.BlockSpec((N_GRAPHS, 2 * EMB), lambda i: (0, 0)),
        ],
        out_shape=[
            jax.ShapeDtypeStruct((N_EDGES // 2, 2 * EMB), F32),
            jax.ShapeDtypeStruct((GF, 1, RF), F32),
            jax.ShapeDtypeStruct((GF, 1, RF), F32),
            jax.ShapeDtypeStruct((N_GRAPHS, 2 * EMB), F32),
        ],
    )(ea_t, ea_t, ga, gb, bb3, bb3, we1, be1, we, beu, du, we2r, sc)


def _node_body(v0_ref, sp_ref, cnt_ref, bt_ref, u0_ref, wvv_ref, wve_ref,
               wvu_ref, bvu_ref, wv2_ref, eacc_ref, wuv_ref, wue_ref,
               wuu_ref, buu_ref, wu2_ref, sc_ref,
               v0p_ref, sa_ref, sb_ref, u0p_ref, vacc_ref):
    i = pl.program_id(0)
    cnt = cnt_ref[0, 0, :] + cnt_ref[0, 1, :]
    inv = 1.0 / jnp.maximum(cnt, 1.0)
    e_to_v = (sp_ref[0] + sp_ref[1]) * inv[:, None]
    btv = bt_ref[0, 0, :]
    oh = (btv[:, None] == lax.broadcasted_iota(jnp.int32, (RN, N_GRAPHS), 1)
          ).astype(F32)
    ub = jnp.dot(u0_ref[...], wvu_ref[...], preferred_element_type=F32)
    v0 = v0_ref[...]
    v1 = _sp(jnp.dot(v0, wvv_ref[...], preferred_element_type=F32)
             + jnp.dot(e_to_v, wve_ref[...], preferred_element_type=F32)
             + jnp.dot(oh, ub, preferred_element_type=F32) + bvu_ref[...])
    v0p = _sp(jnp.sum((v1 + v0) * wv2_ref[...], axis=1) + sc_ref[0, 0])
    v0p_ref[0, 0, :] = v0p
    sa_ref[0, 0, :] = sc_ref[0, 2] * v0p
    sb_ref[0, 0, :] = sc_ref[0, 3] * v0p

    @pl.when(i == 0)
    def _():
        vacc_ref[...] = jnp.zeros_like(vacc_ref)

    seg = lax.dot_general(oh, v1, (((0,), (0,)), ((), ())),
                          preferred_element_type=F32)
    vacc_ref[:, :EMB] += seg
    vacc_ref[:, EMB:] += jnp.sum(oh, axis=0)[:, None]

    @pl.when(i == pl.num_programs(0) - 1)
    def _():
        v_to_u = vacc_ref[:, :EMB] / jnp.maximum(vacc_ref[:, EMB:], 1.0)
        e_to_u = eacc_ref[:, :EMB] / jnp.maximum(eacc_ref[:, EMB:], 1.0)
        u0 = u0_ref[...]
        u1 = _sp(jnp.dot(v_to_u, wuv_ref[...], preferred_element_type=F32)
                 + jnp.dot(e_to_u, wue_ref[...], preferred_element_type=F32)
                 + jnp.dot(u0, wuu_ref[...], preferred_element_type=F32)
                 + buu_ref[...])
        u0p_ref[0, :] = _sp(jnp.sum((u1 + u0) * wu2_ref[...], axis=1)
                            + sc_ref[0, 1])


def _node_call(v0, sp_, cntp, bt3, u0, wvv, wve, wvu, bvu, wv2r, eacc,
               wuv, wue, wuu, buu, wu2r, sc):
    full = lambda shape: pl.BlockSpec(shape, lambda i: (0,) * len(shape))
    return pl.pallas_call(
        _node_body,
        grid=(GN,),
        in_specs=[
            pl.BlockSpec((RN, EMB), lambda i: (i, 0)),
            pl.BlockSpec((2, RN, EMB), lambda i: (0, i, 0)),
            pl.BlockSpec((1, 2, RN), lambda i: (i, 0, 0)),
            pl.BlockSpec((1, 1, RN), lambda i: (i, 0, 0)),
            full((N_GRAPHS, EMB)), full((EMB, EMB)), full((EMB, EMB)),
            full((EMB, EMB)), full((1, EMB)), full((1, EMB)),
            full((N_GRAPHS, 2 * EMB)), full((EMB, EMB)), full((EMB, EMB)),
            full((EMB, EMB)), full((1, EMB)), full((1, EMB)), full((1, 8)),
        ],
        out_specs=[
            pl.BlockSpec((1, 1, RN), lambda i: (i, 0, 0)),
            pl.BlockSpec((1, 1, RN), lambda i: (i, 0, 0)),
            pl.BlockSpec((1, 1, RN), lambda i: (i, 0, 0)),
            pl.BlockSpec((1, N_GRAPHS), lambda i: (0, 0)),
        ],
        out_shape=[
            jax.ShapeDtypeStruct((GN, 1, RN), F32),
            jax.ShapeDtypeStruct((GN, 1, RN), F32),
            jax.ShapeDtypeStruct((GN, 1, RN), F32),
            jax.ShapeDtypeStruct((1, N_GRAPHS), F32),
        ],
        scratch_shapes=[pltpu.VMEM((N_GRAPHS, 2 * EMB), F32)],
    )(v0, sp_, cntp, bt3, u0, wvv, wve, wvu, bvu, wv2r, eacc,
      wuv, wue, wuu, buu, wu2r, sc)


E2R = 2500
E2B = 2500


def _edge2_body(ts_ref, td_ref, e0p_ref, bb_ref, u0p_ref, sc_ref, e1p_ref,
                eo_ref):
    bbv = bb_ref[...]
    u_e = jnp.zeros((E2B, 128), F32)
    for g in range(N_GRAPHS):
        u_e = u_e + jnp.where(bbv == g, u0p_ref[0, g], 0.0)
    e0p = e0p_ref[...]
    we, wu, beu2 = sc_ref[0, 0], sc_ref[0, 1], sc_ref[0, 2]
    e1p = _sp(ts_ref[...] + td_ref[...] + we * e0p + wu * u_e + beu2)
    e1p_ref[...] = e1p
    eo_ref[...] = e1p + e0p


def _edge2_call(ts2, td2, e0p2, bb2, u0p, sc):
    full = lambda shape: pl.BlockSpec(shape, lambda i: (0,) * len(shape))
    blk = lambda: pl.BlockSpec((E2B, 128), lambda i: (i, 0))
    return pl.pallas_call(
        _edge2_body,
        grid=(E2R // E2B,),
        in_specs=[blk(), blk(), blk(), blk(), full((1, N_GRAPHS)),
                  full((1, 8))],
        out_specs=[blk(), blk()],
        out_shape=[
            jax.ShapeDtypeStruct((E2R, 128), F32),
            jax.ShapeDtypeStruct((E2R, 128), F32),
        ],
    )(ts2, td2, e0p2, bb2, u0p, sc)


def _node2_body(v0p_ref, s2_ref, cnt_ref, bt_ref, u0p_ref, sc_ref, vo_ref):
    cnt = cnt_ref[0, 0, :] + cnt_ref[0, 1, :]
    inv = 1.0 / jnp.maximum(cnt, 1.0)
    e_to_vp = (s2_ref[0, 0, :] + s2_ref[0, 1, :]) * inv
    btv = bt_ref[0, 0, :]
    oh = (btv[:, None] == lax.broadcasted_iota(jnp.int32, (RN, N_GRAPHS), 1)
          ).astype(F32)
    u_n = jnp.sum(oh * u0p_ref[...], axis=1)
    wvv, wve, wvu, bv2u = (sc_ref[0, 0], sc_ref[0, 1], sc_ref[0, 2],
                           sc_ref[0, 3])
    v0p = v0p_ref[0, 0, :]
    v1p = _sp(wvv * v0p + wve * e_to_vp + wvu * u_n + bv2u)
    vo_ref[0, 0, :] = v1p + v0p


def _node2_call(v0p3, s2p, cntp, bt3, u0p, sc):
    full = lambda shape: pl.BlockSpec(shape, lambda i: (0,) * len(shape))
    return pl.pallas_call(
        _node2_body,
        grid=(GN,),
        in_specs=[
            pl.BlockSpec((1, 1, RN), lambda i: (i, 0, 0)),
            pl.BlockSpec((1, 2, RN), lambda i: (i, 0, 0)),
            pl.BlockSpec((1, 2, RN), lambda i: (i, 0, 0)),
            pl.BlockSpec((1, 1, RN), lambda i: (i, 0, 0)),
            full((1, N_GRAPHS)), full((1, 8)),
        ],
        out_specs=pl.BlockSpec((1, 1, RN), lambda i: (i, 0, 0)),
        out_shape=jax.ShapeDtypeStruct((GN, 1, RN), F32),
    )(v0p3, s2p, cntp, bt3, u0p, sc)


def _sc_gather_body(a_hbm, b_hbm, src_hbm, dst_hbm, ga_hbm, gb_hbm,
                    idx_v, buf_v, sem):
    wid = lax.axis_index("s") * 2 + lax.axis_index("c")
    half = wid // 16
    rbase = (wid % 16) * EPW
    col = half * EMB
    base = wid * EPW
    for kk in range(EPW // SCC):
        off = base + kk * SCC
        ro = rbase + kk * SCC
        pltpu.sync_copy(src_hbm.at[pl.ds(off, SCC)], idx_v)
        pltpu.async_copy(a_hbm.at[idx_v], buf_v, sem).wait()
        pltpu.sync_copy(buf_v, ga_hbm.at[pl.ds(ro, SCC), pl.ds(col, EMB)])
        pltpu.sync_copy(dst_hbm.at[pl.ds(off, SCC)], idx_v)
        pltpu.async_copy(b_hbm.at[idx_v], buf_v, sem).wait()
        pltpu.sync_copy(buf_v, gb_hbm.at[pl.ds(ro, SCC), pl.ds(col, EMB)])


_sc_gather = pl.kernel(
    _sc_gather_body,
    mesh=plsc.VectorSubcoreMesh(**_SC_MESH),
    compiler_params=_SC_PARAMS,
    out_type=[jax.ShapeDtypeStruct((N_EDGES // 2, 2 * EMB), F32),
              jax.ShapeDtypeStruct((N_EDGES // 2, 2 * EMB), F32)],
    scratch_types=[pltpu.VMEM((SCC,), jnp.int32),
                   pltpu.VMEM((SCC, EMB), F32),
                   pltpu.SemaphoreType.DMA],
)


SCF = 1000


def _sc_scatter_body(e1_hbm, dst_hbm, z2_hbm, z1_hbm, one_hbm, s_hbm,
                     cnt_hbm, idx_v, buf_v, ones_v, acc, acc1, sem):
    c = lax.axis_index("c")
    s = lax.axis_index("s")
    wid = s * 2 + c
    rpt = N_NODES // 16
    rpt1 = NP1 // 16
    half = N_EDGES // 2
    pltpu.sync_copy(z2_hbm.at[pl.ds(s * rpt, rpt)],
                    acc.at[pl.ds(s * rpt, rpt)])
    pltpu.sync_copy(z1_hbm.at[pl.ds(s * rpt1, rpt1)],
                    acc1.at[pl.ds(s * rpt1, rpt1)])
    pltpu.sync_copy(one_hbm, ones_v)
    plsc.subcore_barrier()
    fbase = wid * (EPW // 2)
    for kk in range(EPW // 2 // SCF):
        ro = fbase + kk * SCF
        for col, ebase in ((0, 0), (EMB, half)):
            pltpu.sync_copy(dst_hbm.at[pl.ds(ebase + ro, SCF)], idx_v)
            pltpu.sync_copy(e1_hbm.at[pl.ds(ro, SCF), pl.ds(col, EMB)],
                            buf_v)
            pltpu.sync_copy(buf_v, acc.at[idx_v], add=True)
            pltpu.sync_copy(ones_v, acc1.at[idx_v], add=True)
    plsc.subcore_barrier()
    pltpu.sync_copy(acc.at[pl.ds(s * rpt, rpt)],
                    s_hbm.at[c, pl.ds(s * rpt, rpt)])
    pltpu.sync_copy(acc1.at[pl.ds(s * rpt1, rpt1)],
                    cnt_hbm.at[c, pl.ds(s * rpt1, rpt1)])


_sc_scatter = pl.kernel(
    _sc_scatter_body,
    mesh=plsc.VectorSubcoreMesh(**_SC_MESH),
    compiler_params=_SC_PARAMS,
    out_type=[jax.ShapeDtypeStruct((2, N_NODES, EMB), F32),
              jax.ShapeDtypeStruct((2, NP1), F32)],
    scratch_types=[pltpu.VMEM((SCF,), jnp.int32),
                   pltpu.VMEM((SCF, EMB), F32),
                   pltpu.VMEM((SCF,), F32),
                   pltpu.VMEM_SHARED((N_NODES, EMB), F32),
                   pltpu.VMEM_SHARED((NP1,), F32),
                   pltpu.SemaphoreType.DMA],
)


def _sc_gather1_body(sa_hbm, sb_hbm, src_hbm, dst_hbm, ts_hbm, td_hbm,
                     idx_v, val_v, sem):
    wid = lax.axis_index("s") * 2 + lax.axis_index("c")
    base = wid * EPW
    for kk in range(EPW // SCC):
        off = base + kk * SCC
        pltpu.sync_copy(src_hbm.at[pl.ds(off, SCC)], idx_v)
        pltpu.async_copy(sa_hbm.at[idx_v], val_v, sem).wait()
        pltpu.sync_copy(val_v, ts_hbm.at[pl.ds(off, SCC)])
        pltpu.sync_copy(dst_hbm.at[pl.ds(off, SCC)], idx_v)
        pltpu.async_copy(sb_hbm.at[idx_v], val_v, sem).wait()
        pltpu.sync_copy(val_v, td_hbm.at[pl.ds(off, SCC)])


_sc_gather1 = pl.kernel(
    _sc_gather1_body,
    mesh=plsc.VectorSubcoreMesh(**_SC_MESH),
    compiler_params=_SC_PARAMS,
    out_type=[jax.ShapeDtypeStruct((N_EDGES,), F32),
              jax.ShapeDtypeStruct((N_EDGES,), F32)],
    scratch_types=[pltpu.VMEM((SCC,), jnp.int32),
                   pltpu.VMEM((SCC,), F32),
                   pltpu.SemaphoreType.DMA],
)


def _sc_scatter1_body(ep_hbm, dst_hbm, z1_hbm, s2_hbm, idx_v, val_v, acc1,
                      sem):
    c = lax.axis_index("c")
    s = lax.axis_index("s")
    wid = s * 2 + c
    rpt1 = NP1 // 16
    pltpu.sync_copy(z1_hbm.at[pl.ds(s * rpt1, rpt1)],
                    acc1.at[pl.ds(s * rpt1, rpt1)])
    plsc.subcore_barrier()
    base = wid * EPW
    for kk in range(EPW // SCC):
        off = base + kk * SCC
        pltpu.sync_copy(dst_hbm.at[pl.ds(off, SCC)], idx_v)
        pltpu.sync_copy(ep_hbm.at[pl.ds(off, SCC)], val_v)
        pltpu.sync_copy(val_v, acc1.at[idx_v], add=True)
    plsc.subcore_barrier()
    pltpu.sync_copy(acc1.at[pl.ds(s * rpt1, rpt1)],
                    s2_hbm.at[c, pl.ds(s * rpt1, rpt1)])


_sc_scatter1 = pl.kernel(
    _sc_scatter1_body,
    mesh=plsc.VectorSubcoreMesh(**_SC_MESH),
    compiler_params=_SC_PARAMS,
    out_type=jax.ShapeDtypeStruct((2, NP1), F32),
    scratch_types=[pltpu.VMEM((SCC,), jnp.int32),
                   pltpu.VMEM((SCC,), F32),
                   pltpu.VMEM_SHARED((NP1,), F32),
                   pltpu.SemaphoreType.DMA],
)


def kernel(x, edge_index, edge_attr, state, batch, bond_batch, params1,
           params2):
    p1, p2 = params1, params2
    src = edge_index[0].astype(jnp.int32)
    dst = edge_index[1].astype(jnp.int32)
    batch_i = batch.astype(jnp.int32)
    bb_i = bond_batch.astype(jnp.int32)

    row = lambda v: v.reshape(1, -1)

    W = p1["We_upd"]
    v0, a, b, u0, du = _pre_call(
        x, p1["Wv_pre"], row(p1["bv_pre"]), W[:EMB], W[EMB:2 * EMB],
        state, p1["Wu_pre"], row(p1["bu_pre"]), W[3 * EMB:])

    ga, gb = _sc_gather(a, b, src, dst)

    sc_c = jnp.zeros((1, 8), F32).at[0, 0].set(p2["be_pre"][0])
    bb3 = bb_i.reshape(2 * GF, 1, RF)
    e1f, e0p_lo3, e0p_hi3, eacc = _edge_call(
        edge_attr.T, ga, gb, bb3, p1["We_pre"], row(p1["be_pre"]),
        W[2 * EMB:3 * EMB], row(p1["be_upd"]), du, row(p2["We_pre"][:, 0]),
        sc_c)

    half = N_EDGES // 2
    z2 = jnp.zeros((N_NODES, EMB), F32)
    z1 = jnp.zeros((NP1,), F32)
    one = jnp.ones((SCF,), F32)
    sp_, cntp = _sc_scatter(e1f, dst, z2, z1, one)
    cntp3 = cntp[:, :N_NODES].reshape(2, GN, RN).transpose(1, 0, 2)

    Wv = p1["Wv_upd"]
    Wu = p1["Wu_upd"]
    W2 = p2["We_upd"][:, 0]
    sc_e = (jnp.zeros((1, 8), F32).at[0, 0].set(p2["bv_pre"][0])
            .at[0, 1].set(p2["bu_pre"][0])
            .at[0, 2].set(W2[0]).at[0, 3].set(W2[1]))
    bt3 = batch_i.reshape(GN, 1, RN)
    v0p3, sa3, sb3, u0p = _node_call(
        v0, sp_, cntp3, bt3, u0, Wv[:EMB], Wv[EMB:2 * EMB], Wv[2 * EMB:],
        row(p1["bv_upd"]), row(p2["Wv_pre"][:, 0]), eacc,
        Wu[:EMB], Wu[EMB:2 * EMB], Wu[2 * EMB:], row(p1["bu_upd"]),
        row(p2["Wu_pre"][:, 0]), sc_e)

    ts, td = _sc_gather1(sa3.reshape(N_NODES), sb3.reshape(N_NODES),
                         src, dst)

    sc_g = (jnp.zeros((1, 8), F32).at[0, 0].set(W2[2]).at[0, 1].set(W2[3])
            .at[0, 2].set(p2["be_upd"][0]))
    e0p_f = jnp.concatenate([e0p_lo3.reshape(half), e0p_hi3.reshape(half)])
    e1p2, eo2 = _edge2_call(ts.reshape(E2R, 128),
                            td.reshape(E2R, 128),
                            e0p_f.reshape(E2R, 128),
                            bb_i.reshape(E2R, 128), u0p, sc_g)

    s2p = _sc_scatter1(e1p2.reshape(N_EDGES), dst, z1)
    s2p3 = s2p[:, :N_NODES].reshape(2, GN, RN).transpose(1, 0, 2)

    Wv2 = p2["Wv_upd"][:, 0]
    sc_i = (jnp.zeros((1, 8), F32).at[0, 0].set(Wv2[0]).at[0, 1].set(Wv2[1])
            .at[0, 2].set(Wv2[2]).at[0, 3].set(p2["bv_upd"][0]))
    vo3 = _node2_call(v0p3, s2p3, cntp3, bt3, u0p, sc_i)

    v_out = vo3.reshape(N_NODES, 1)
    e_out = eo2.reshape(N_EDGES, 1)
    return (v_out, e_out, edge_index)


# speedup vs baseline: 1.5048x; 1.0187x over previous
import functools

import jax
import jax.numpy as jnp
from jax import lax
from jax.experimental import pallas as pl
from jax.experimental.pallas import tpu as pltpu
from jax.experimental.pallas import tpu_sc as plsc

N_NODES = 10000
N_EDGES = 320000
N_GRAPHS = 16
EMB = 64

RN = 1000
GN = N_NODES // RN
RE = 6400
GE = N_EDGES // RE
RE2 = 8000
GE2 = N_EDGES // RE2

F32 = jnp.float32

NW = 32
EPW = N_EDGES // NW
SCC = 1000
NP1 = 10240
_SC_PARAMS = pltpu.CompilerParams(use_tc_tiling_on_sc=False)
_SC_MESH = dict(core_axis_name="c", subcore_axis_name="s")


_LOG2E = 1.4426950408889634
_LN2 = 0.6931471805599453


def _sp(x):
    return _LN2 * jnp.log2(1.0 + jnp.exp2(-_LOG2E * jnp.abs(x))) \
        + jnp.maximum(x, 0.0)


def _pre_body(x_ref, wv_ref, bv_ref, ws_ref, wd_ref, st_ref, wu_ref, bu_ref,
              wue_ref, v0_ref, a_ref, b_ref, u0_ref, du_ref):
    i = pl.program_id(0)
    v0 = _sp(jnp.dot(x_ref[...], wv_ref[...], preferred_element_type=F32)
             + bv_ref[...])
    v0_ref[...] = v0
    a_ref[...] = jnp.dot(v0, ws_ref[...], preferred_element_type=F32)
    b_ref[...] = jnp.dot(v0, wd_ref[...], preferred_element_type=F32)

    @pl.when(i == 0)
    def _():
        u0 = _sp(jnp.dot(st_ref[...], wu_ref[...], preferred_element_type=F32)
                 + bu_ref[...])
        u0_ref[...] = u0
        du_ref[...] = jnp.dot(u0, wue_ref[...], preferred_element_type=F32)


def _pre_call(x, wv, bv, ws, wd, st, wu, bu, wue):
    d_node = x.shape[1]
    d_u = st.shape[1]
    full = lambda shape: pl.BlockSpec(shape, lambda i: (0,) * len(shape))
    return pl.pallas_call(
        _pre_body,
        grid=(GN,),
        in_specs=[
            pl.BlockSpec((RN, d_node), lambda i: (i, 0)),
            full((d_node, EMB)), full((1, EMB)), full((EMB, EMB)),
            full((EMB, EMB)), full((N_GRAPHS, d_u)), full((d_u, EMB)),
            full((1, EMB)), full((EMB, EMB)),
        ],
        out_specs=[
            pl.BlockSpec((RN, EMB), lambda i: (i, 0)),
            pl.BlockSpec((RN, EMB), lambda i: (i, 0)),
            pl.BlockSpec((RN, EMB), lambda i: (i, 0)),
            pl.BlockSpec((N_GRAPHS, EMB), lambda i: (0, 0)),
            pl.BlockSpec((N_GRAPHS, EMB), lambda i: (0, 0)),
        ],
        out_shape=[
            jax.ShapeDtypeStruct((N_NODES, EMB), F32),
            jax.ShapeDtypeStruct((N_NODES, EMB), F32),
            jax.ShapeDtypeStruct((N_NODES, EMB), F32),
            jax.ShapeDtypeStruct((N_GRAPHS, EMB), F32),
            jax.ShapeDtypeStruct((N_GRAPHS, EMB), F32),
        ],
    )(x, wv, bv, ws, wd, st, wu, bu, wue)


RF = RE // 2
GF = N_EDGES // 2 // RF


def _edge_half(ea_blk, bb_row, g, we1_ref, be1_ref, we_ref, beu_ref, du_ref,
               we2_ref, sc_ref):
    e0 = _sp(lax.dot_general(ea_blk, we1_ref[...], (((0,), (0,)), ((), ())),
                             preferred_element_type=F32) + be1_ref[...])
    oh = (bb_row[:, None]
          == lax.broadcasted_iota(jnp.int32, (RF, N_GRAPHS), 1)).astype(F32)
    dub = jnp.dot(oh, du_ref[...], preferred_element_type=F32)
    e1 = _sp(g + jnp.dot(e0, we_ref[...], preferred_element_type=F32)
             + dub + beu_ref[...])
    eB = e1 + e0
    e0p = _sp(jnp.sum(eB * we2_ref[...], axis=1) + sc_ref[0, 0])
    e1c = jnp.concatenate([e1, jnp.ones((RF, EMB), F32)], axis=1)
    seg = lax.dot_general(oh, e1c, (((0,), (0,)), ((), ())),
                          preferred_element_type=F32)
    return e1, e0p, seg


def _edge_body(ea_lo_ref, ea_hi_ref, ga_ref, gb_ref, bb_lo_ref, bb_hi_ref,
               we1_ref, be1_ref, we_ref, beu_ref, du_ref, we2_ref, sc_ref,
               e1_ref, e0p_lo_ref, e0p_hi_ref, acc_ref):
    i = pl.program_id(0)
    gab = ga_ref[...] + gb_ref[...]
    e1_lo, e0p_lo, seg_lo = _edge_half(
        ea_lo_ref[...], bb_lo_ref[0, 0, :], gab[:, :EMB], we1_ref, be1_ref,
        we_ref, beu_ref, du_ref, we2_ref, sc_ref)
    e1_hi, e0p_hi, seg_hi = _edge_half(
        ea_hi_ref[...], bb_hi_ref[0, 0, :], gab[:, EMB:], we1_ref, be1_ref,
        we_ref, beu_ref, du_ref, we2_ref, sc_ref)
    e1_ref[...] = jnp.concatenate([e1_lo, e1_hi], axis=1)
    e0p_lo_ref[0, 0, :] = e0p_lo
    e0p_hi_ref[0, 0, :] = e0p_hi

    @pl.when(i == 0)
    def _():
        acc_ref[...] = jnp.zeros_like(acc_ref)

    acc_ref[...] += seg_lo + seg_hi


def _edge_call(ea_t, ga, gb, bb3, we1, be1, we, beu, du, we2r, sc):
    d_edge = ea_t.shape[0]
    full = lambda shape: pl.BlockSpec(shape, lambda i: (0,) * len(shape))
    return pl.pallas_call(
        _edge_body,
        grid=(GF,),
        in_specs=[
            pl.BlockSpec((d_edge, RF), lambda i: (0, i)),
            pl.BlockSpec((d_edge, RF), lambda i: (0, i + GF)),
            pl.BlockSpec((RF, 2 * EMB), lambda i: (i, 0)),
            pl.BlockSpec((RF, 2 * EMB), lambda i: (i, 0)),
            pl.BlockSpec((1, 1, RF), lambda i: (i, 0, 0)),
            pl.BlockSpec((1, 1, RF), lambda i: (i + GF, 0, 0)),
            full((d_edge, EMB)), full((1, EMB)), full((EMB, EMB)),
            full((1, EMB)), full((N_GRAPHS, EMB)), full((1, EMB)),
            full((1, 8)),
        ],
        out_specs=[
            pl.BlockSpec((RF, 2 * EMB), lambda i: (i, 0)),
            pl.BlockSpec((1, 1, RF), lambda i: (i, 0, 0)),
            pl.BlockSpec((1, 1, RF), lambda i: (i, 0, 0)),
            pl.BlockSpec((N_GRAPHS, 2 * EMB), lambda i: (0, 0)),
        ],
        out_shape=[
            jax.ShapeDtypeStruct((N_EDGES // 2, 2 * EMB), F32),
            jax.ShapeDtypeStruct((GF, 1, RF), F32),
            jax.ShapeDtypeStruct((GF, 1, RF), F32),
            jax.ShapeDtypeStruct((N_GRAPHS, 2 * EMB), F32),
        ],
    )(ea_t, ea_t, ga, gb, bb3, bb3, we1, be1, we, beu, du, we2r, sc)


def _node_body(v0_ref, sp_ref, cnt_ref, bt_ref, u0_ref, wvv_ref, wve_ref,
               wvu_ref, bvu_ref, wv2_ref, eacc_ref, wuv_ref, wue_ref,
               wuu_ref, buu_ref, wu2_ref, sc_ref,
               v0p_ref, sa_ref, sb_ref, u0p_ref, vacc_ref):
    i = pl.program_id(0)
    cnt = cnt_ref[0, 0, :] + cnt_ref[0, 1, :]
    inv = 1.0 / jnp.maximum(cnt, 1.0)
    e_to_v = (sp_ref[0] + sp_ref[1]) * inv[:, None]
    btv = bt_ref[0, 0, :]
    oh = (btv[:, None] == lax.broadcasted_iota(jnp.int32, (RN, N_GRAPHS), 1)
          ).astype(F32)
    ub = jnp.dot(u0_ref[...], wvu_ref[...], preferred_element_type=F32)
    v0 = v0_ref[...]
    v1 = _sp(jnp.dot(v0, wvv_ref[...], preferred_element_type=F32)
             + jnp.dot(e_to_v, wve_ref[...], preferred_element_type=F32)
             + jnp.dot(oh, ub, preferred_element_type=F32) + bvu_ref[...])
    v0p = _sp(jnp.sum((v1 + v0) * wv2_ref[...], axis=1) + sc_ref[0, 0])
    v0p_ref[0, 0, :] = v0p
    sa_ref[0, 0, :] = sc_ref[0, 2] * v0p
    sb_ref[0, 0, :] = sc_ref[0, 3] * v0p

    @pl.when(i == 0)
    def _():
        vacc_ref[...] = jnp.zeros_like(vacc_ref)

    seg = lax.dot_general(oh, v1, (((0,), (0,)), ((), ())),
                          preferred_element_type=F32)
    vacc_ref[:, :EMB] += seg
    vacc_ref[:, EMB:] += jnp.sum(oh, axis=0)[:, None]

    @pl.when(i == pl.num_programs(0) - 1)
    def _():
        v_to_u = vacc_ref[:, :EMB] / jnp.maximum(vacc_ref[:, EMB:], 1.0)
        e_to_u = eacc_ref[:, :EMB] / jnp.maximum(eacc_ref[:, EMB:], 1.0)
        u0 = u0_ref[...]
        u1 = _sp(jnp.dot(v_to_u, wuv_ref[...], preferred_element_type=F32)
                 + jnp.dot(e_to_u, wue_ref[...], preferred_element_type=F32)
                 + jnp.dot(u0, wuu_ref[...], preferred_element_type=F32)
                 + buu_ref[...])
        u0p_ref[0, :] = _sp(jnp.sum((u1 + u0) * wu2_ref[...], axis=1)
                            + sc_ref[0, 1])


def _node_call(v0, sp_, cntp, bt3, u0, wvv, wve, wvu, bvu, wv2r, eacc,
               wuv, wue, wuu, buu, wu2r, sc):
    full = lambda shape: pl.BlockSpec(shape, lambda i: (0,) * len(shape))
    return pl.pallas_call(
        _node_body,
        grid=(GN,),
        in_specs=[
            pl.BlockSpec((RN, EMB), lambda i: (i, 0)),
            pl.BlockSpec((2, RN, EMB), lambda i: (0, i, 0)),
            pl.BlockSpec((1, 2, RN), lambda i: (i, 0, 0)),
            pl.BlockSpec((1, 1, RN), lambda i: (i, 0, 0)),
            full((N_GRAPHS, EMB)), full((EMB, EMB)), full((EMB, EMB)),
            full((EMB, EMB)), full((1, EMB)), full((1, EMB)),
            full((N_GRAPHS, 2 * EMB)), full((EMB, EMB)), full((EMB, EMB)),
            full((EMB, EMB)), full((1, EMB)), full((1, EMB)), full((1, 8)),
        ],
        out_specs=[
            pl.BlockSpec((1, 1, RN), lambda i: (i, 0, 0)),
            pl.BlockSpec((1, 1, RN), lambda i: (i, 0, 0)),
            pl.BlockSpec((1, 1, RN), lambda i: (i, 0, 0)),
            pl.BlockSpec((1, N_GRAPHS), lambda i: (0, 0)),
        ],
        out_shape=[
            jax.ShapeDtypeStruct((GN, 1, RN), F32),
            jax.ShapeDtypeStruct((GN, 1, RN), F32),
            jax.ShapeDtypeStruct((GN, 1, RN), F32),
            jax.ShapeDtypeStruct((1, N_GRAPHS), F32),
        ],
        scratch_shapes=[pltpu.VMEM((N_GRAPHS, 2 * EMB), F32)],
    )(v0, sp_, cntp, bt3, u0, wvv, wve, wvu, bvu, wv2r, eacc,
      wuv, wue, wuu, buu, wu2r, sc)


E2R = 2500
E2B = 2500


def _edge2_body(ts_ref, td_ref, e0p_ref, bb_ref, u0p_ref, sc_ref, e1p_ref,
                eo_ref):
    bbv = bb_ref[...]
    u_e = jnp.zeros((E2B, 128), F32)
    for g in range(N_GRAPHS):
        u_e = u_e + jnp.where(bbv == g, u0p_ref[0, g], 0.0)
    e0p = e0p_ref[...]
    we, wu, beu2 = sc_ref[0, 0], sc_ref[0, 1], sc_ref[0, 2]
    e1p = _sp(ts_ref[...] + td_ref[...] + we * e0p + wu * u_e + beu2)
    e1p_ref[...] = e1p
    eo_ref[...] = e1p + e0p


def _edge2_call(ts2, td2, e0p2, bb2, u0p, sc):
    full = lambda shape: pl.BlockSpec(shape, lambda i: (0,) * len(shape))
    blk = lambda: pl.BlockSpec((E2B, 128), lambda i: (i, 0))
    return pl.pallas_call(
        _edge2_body,
        grid=(E2R // E2B,),
        in_specs=[blk(), blk(), blk(), blk(), full((1, N_GRAPHS)),
                  full((1, 8))],
        out_specs=[blk(), blk()],
        out_shape=[
            jax.ShapeDtypeStruct((E2R, 128), F32),
            jax.ShapeDtypeStruct((E2R, 128), F32),
        ],
    )(ts2, td2, e0p2, bb2, u0p, sc)


def _node2_body(v0p_ref, s2_ref, cnt_ref, bt_ref, u0p_ref, sc_ref, vo_ref):
    cnt = cnt_ref[0, 0, :] + cnt_ref[0, 1, :]
    inv = 1.0 / jnp.maximum(cnt, 1.0)
    e_to_vp = (s2_ref[0, 0, :] + s2_ref[0, 1, :]) * inv
    btv = bt_ref[0, 0, :]
    oh = (btv[:, None] == lax.broadcasted_iota(jnp.int32, (RN, N_GRAPHS), 1)
          ).astype(F32)
    u_n = jnp.sum(oh * u0p_ref[...], axis=1)
    wvv, wve, wvu, bv2u = (sc_ref[0, 0], sc_ref[0, 1], sc_ref[0, 2],
                           sc_ref[0, 3])
    v0p = v0p_ref[0, 0, :]
    v1p = _sp(wvv * v0p + wve * e_to_vp + wvu * u_n + bv2u)
    vo_ref[0, 0, :] = v1p + v0p


def _node2_call(v0p3, s2p, cntp, bt3, u0p, sc):
    full = lambda shape: pl.BlockSpec(shape, lambda i: (0,) * len(shape))
    return pl.pallas_call(
        _node2_body,
        grid=(GN,),
        in_specs=[
            pl.BlockSpec((1, 1, RN), lambda i: (i, 0, 0)),
            pl.BlockSpec((1, 2, RN), lambda i: (i, 0, 0)),
            pl.BlockSpec((1, 2, RN), lambda i: (i, 0, 0)),
            pl.BlockSpec((1, 1, RN), lambda i: (i, 0, 0)),
            full((1, N_GRAPHS)), full((1, 8)),
        ],
        out_specs=pl.BlockSpec((1, 1, RN), lambda i: (i, 0, 0)),
        out_shape=jax.ShapeDtypeStruct((GN, 1, RN), F32),
    )(v0p3, s2p, cntp, bt3, u0p, sc)


def _sc_gather_body(a_hbm, b_hbm, src_hbm, dst_hbm, ga_hbm, gb_hbm,
                    idx0_v, idx1_v, buf0_v, buf1_v, sem_g, sem_w):
    wid = lax.axis_index("s") * 2 + lax.axis_index("c")
    half = wid // 16
    rbase = (wid % 16) * EPW
    col = half * EMB
    base = wid * EPW
    idx = (idx0_v, idx1_v)
    buf = (buf0_v, buf1_v)
    nchunks = EPW // SCC
    units = []
    for kk in range(nchunks):
        off = base + kk * SCC
        ro = rbase + kk * SCC
        units.append((src_hbm, a_hbm, off, ga_hbm, ro))
        units.append((dst_hbm, b_hbm, off, gb_hbm, ro))
    gh = [None] * len(units)
    wh = [None] * len(units)
    for u, (i_hbm, t_hbm, off, o_hbm, ro) in enumerate(units):
        if u >= 2:
            wh[u - 2].wait()
        pltpu.sync_copy(i_hbm.at[pl.ds(off, SCC)], idx[u % 2])
        gh[u] = pltpu.async_copy(t_hbm.at[idx[u % 2]], buf[u % 2], sem_g)
        if u >= 1:
            pu = u - 1
            gh[pu].wait()
            _, _, _, po_hbm, pro = units[pu]
            wh[pu] = pltpu.async_copy(
                buf[pu % 2], po_hbm.at[pl.ds(pro, SCC), pl.ds(col, EMB)],
                sem_w)
    lu = len(units) - 1
    gh[lu].wait()
    wh[lu] = pltpu.async_copy(
        buf[lu % 2], units[lu][3].at[pl.ds(units[lu][4], SCC),
                                     pl.ds(col, EMB)], sem_w)
    wh[lu - 1].wait()
    wh[lu].wait()


_sc_gather = pl.kernel(
    _sc_gather_body,
    mesh=plsc.VectorSubcoreMesh(**_SC_MESH),
    compiler_params=_SC_PARAMS,
    out_type=[jax.ShapeDtypeStruct((N_EDGES // 2, 2 * EMB), F32),
              jax.ShapeDtypeStruct((N_EDGES // 2, 2 * EMB), F32)],
    scratch_types=[pltpu.VMEM((SCC,), jnp.int32),
                   pltpu.VMEM((SCC,), jnp.int32),
                   pltpu.VMEM((SCC, EMB), F32),
                   pltpu.VMEM((SCC, EMB), F32),
                   pltpu.SemaphoreType.DMA,
                   pltpu.SemaphoreType.DMA],
)


SCF = 1000


def _sc_scatter_body(e1_hbm, dst_hbm, z2_hbm, z1_hbm, one_hbm, s_hbm,
                     cnt_hbm, idx_v, buf_v, ones_v, acc, acc1, sem):
    c = lax.axis_index("c")
    s = lax.axis_index("s")
    wid = s * 2 + c
    rpt = N_NODES // 16
    rpt1 = NP1 // 16
    half = N_EDGES // 2
    pltpu.sync_copy(z2_hbm.at[pl.ds(s * rpt, rpt)],
                    acc.at[pl.ds(s * rpt, rpt)])
    pltpu.sync_copy(z1_hbm.at[pl.ds(s * rpt1, rpt1)],
                    acc1.at[pl.ds(s * rpt1, rpt1)])
    pltpu.sync_copy(one_hbm, ones_v)
    plsc.subcore_barrier()
    fbase = wid * (EPW // 2)
    for kk in range(EPW // 2 // SCF):
        ro = fbase + kk * SCF
        for col, ebase in ((0, 0), (EMB, half)):
            pltpu.sync_copy(dst_hbm.at[pl.ds(ebase + ro, SCF)], idx_v)
            pltpu.sync_copy(e1_hbm.at[pl.ds(ro, SCF), pl.ds(col, EMB)],
                            buf_v)
            pltpu.sync_copy(buf_v, acc.at[idx_v], add=True)
            pltpu.sync_copy(ones_v, acc1.at[idx_v], add=True)
    plsc.subcore_barrier()
    pltpu.sync_copy(acc.at[pl.ds(s * rpt, rpt)],
                    s_hbm.at[c, pl.ds(s * rpt, rpt)])
    pltpu.sync_copy(acc1.at[pl.ds(s * rpt1, rpt1)],
                    cnt_hbm.at[c, pl.ds(s * rpt1, rpt1)])


_sc_scatter = pl.kernel(
    _sc_scatter_body,
    mesh=plsc.VectorSubcoreMesh(**_SC_MESH),
    compiler_params=_SC_PARAMS,
    out_type=[jax.ShapeDtypeStruct((2, N_NODES, EMB), F32),
              jax.ShapeDtypeStruct((2, NP1), F32)],
    scratch_types=[pltpu.VMEM((SCF,), jnp.int32),
                   pltpu.VMEM((SCF, EMB), F32),
                   pltpu.VMEM((SCF,), F32),
                   pltpu.VMEM_SHARED((N_NODES, EMB), F32),
                   pltpu.VMEM_SHARED((NP1,), F32),
                   pltpu.SemaphoreType.DMA],
)


def _sc_gather1_body(sa_hbm, sb_hbm, src_hbm, dst_hbm, ts_hbm, td_hbm,
                     idx_v, val_v, sem):
    wid = lax.axis_index("s") * 2 + lax.axis_index("c")
    base = wid * EPW
    for kk in range(EPW // SCC):
        off = base + kk * SCC
        pltpu.sync_copy(src_hbm.at[pl.ds(off, SCC)], idx_v)
        pltpu.async_copy(sa_hbm.at[idx_v], val_v, sem).wait()
        pltpu.sync_copy(val_v, ts_hbm.at[pl.ds(off, SCC)])
        pltpu.sync_copy(dst_hbm.at[pl.ds(off, SCC)], idx_v)
        pltpu.async_copy(sb_hbm.at[idx_v], val_v, sem).wait()
        pltpu.sync_copy(val_v, td_hbm.at[pl.ds(off, SCC)])


_sc_gather1 = pl.kernel(
    _sc_gather1_body,
    mesh=plsc.VectorSubcoreMesh(**_SC_MESH),
    compiler_params=_SC_PARAMS,
    out_type=[jax.ShapeDtypeStruct((N_EDGES,), F32),
              jax.ShapeDtypeStruct((N_EDGES,), F32)],
    scratch_types=[pltpu.VMEM((SCC,), jnp.int32),
                   pltpu.VMEM((SCC,), F32),
                   pltpu.SemaphoreType.DMA],
)


def _sc_scatter1_body(ep_hbm, dst_hbm, z1_hbm, s2_hbm, idx_v, val_v, acc1,
                      sem):
    c = lax.axis_index("c")
    s = lax.axis_index("s")
    wid = s * 2 + c
    rpt1 = NP1 // 16
    pltpu.sync_copy(z1_hbm.at[pl.ds(s * rpt1, rpt1)],
                    acc1.at[pl.ds(s * rpt1, rpt1)])
    plsc.subcore_barrier()
    base = wid * EPW
    for kk in range(EPW // SCC):
        off = base + kk * SCC
        pltpu.sync_copy(dst_hbm.at[pl.ds(off, SCC)], idx_v)
        pltpu.sync_copy(ep_hbm.at[pl.ds(off, SCC)], val_v)
        pltpu.sync_copy(val_v, acc1.at[idx_v], add=True)
    plsc.subcore_barrier()
    pltpu.sync_copy(acc1.at[pl.ds(s * rpt1, rpt1)],
                    s2_hbm.at[c, pl.ds(s * rpt1, rpt1)])


_sc_scatter1 = pl.kernel(
    _sc_scatter1_body,
    mesh=plsc.VectorSubcoreMesh(**_SC_MESH),
    compiler_params=_SC_PARAMS,
    out_type=jax.ShapeDtypeStruct((2, NP1), F32),
    scratch_types=[pltpu.VMEM((SCC,), jnp.int32),
                   pltpu.VMEM((SCC,), F32),
                   pltpu.VMEM_SHARED((NP1,), F32),
                   pltpu.SemaphoreType.DMA],
)


def kernel(x, edge_index, edge_attr, state, batch, bond_batch, params1,
           params2):
    p1, p2 = params1, params2
    src = edge_index[0].astype(jnp.int32)
    dst = edge_index[1].astype(jnp.int32)
    batch_i = batch.astype(jnp.int32)
    bb_i = bond_batch.astype(jnp.int32)

    row = lambda v: v.reshape(1, -1)

    W = p1["We_upd"]
    v0, a, b, u0, du = _pre_call(
        x, p1["Wv_pre"], row(p1["bv_pre"]), W[:EMB], W[EMB:2 * EMB],
        state, p1["Wu_pre"], row(p1["bu_pre"]), W[3 * EMB:])

    ga, gb = _sc_gather(a, b, src, dst)

    sc_c = jnp.zeros((1, 8), F32).at[0, 0].set(p2["be_pre"][0])
    bb3 = bb_i.reshape(2 * GF, 1, RF)
    e1f, e0p_lo3, e0p_hi3, eacc = _edge_call(
        edge_attr.T, ga, gb, bb3, p1["We_pre"], row(p1["be_pre"]),
        W[2 * EMB:3 * EMB], row(p1["be_upd"]), du, row(p2["We_pre"][:, 0]),
        sc_c)

    half = N_EDGES // 2
    z2 = jnp.zeros((N_NODES, EMB), F32)
    z1 = jnp.zeros((NP1,), F32)
    one = jnp.ones((SCF,), F32)
    sp_, cntp = _sc_scatter(e1f, dst, z2, z1, one)
    cntp3 = cntp[:, :N_NODES].reshape(2, GN, RN).transpose(1, 0, 2)

    Wv = p1["Wv_upd"]
    Wu = p1["Wu_upd"]
    W2 = p2["We_upd"][:, 0]
    sc_e = (jnp.zeros((1, 8), F32).at[0, 0].set(p2["bv_pre"][0])
            .at[0, 1].set(p2["bu_pre"][0])
            .at[0, 2].set(W2[0]).at[0, 3].set(W2[1]))
    bt3 = batch_i.reshape(GN, 1, RN)
    v0p3, sa3, sb3, u0p = _node_call(
        v0, sp_, cntp3, bt3, u0, Wv[:EMB], Wv[EMB:2 * EMB], Wv[2 * EMB:],
        row(p1["bv_upd"]), row(p2["Wv_pre"][:, 0]), eacc,
        Wu[:EMB], Wu[EMB:2 * EMB], Wu[2 * EMB:], row(p1["bu_upd"]),
        row(p2["Wu_pre"][:, 0]), sc_e)

    ts, td = _sc_gather1(sa3.reshape(N_NODES), sb3.reshape(N_NODES),
                         src, dst)

    sc_g = (jnp.zeros((1, 8), F32).at[0, 0].set(W2[2]).at[0, 1].set(W2[3])
            .at[0, 2].set(p2["be_upd"][0]))
    e0p_f = jnp.concatenate([e0p_lo3.reshape(half), e0p_hi3.reshape(half)])
    e1p2, eo2 = _edge2_call(ts.reshape(E2R, 128),
                            td.reshape(E2R, 128),
                            e0p_f.reshape(E2R, 128),
                            bb_i.reshape(E2R, 128), u0p, sc_g)

    s2p = _sc_scatter1(e1p2.reshape(N_EDGES), dst, z1)
    s2p3 = s2p[:, :N_NODES].reshape(2, GN, RN).transpose(1, 0, 2)

    Wv2 = p2["Wv_upd"][:, 0]
    sc_i = (jnp.zeros((1, 8), F32).at[0, 0].set(Wv2[0]).at[0, 1].set(Wv2[1])
            .at[0, 2].set(Wv2[2]).at[0, 3].set(p2["bv_upd"][0]))
    vo3 = _node2_call(v0p3, s2p3, cntp3, bt3, u0p, sc_i)

    v_out = vo3.reshape(N_NODES, 1)
    e_out = eo2.reshape(N_EDGES, 1)
    return (v_out, e_out, edge_index)
